# baseline (speedup 1.0000x reference)
import functools

import jax
import jax.numpy as jnp
from jax import lax
from jax.experimental import pallas as pl
from jax.experimental.pallas import tpu as pltpu
from jax.experimental.pallas import tpu_sc as plsc

N = 10000
NP = 10240
E = 320000
GAMMA = 0.1
K_ITERS = 6
D = 128

NTILES = 16
LANES = 16
PIECES = D // LANES
CHUNK = 128
GB = 8
NB2 = 10
CPT2 = NB2 * GB
EPT2 = CPT2 * CHUNK
NB1 = 2 * NB2
CPT1 = NB1 * GB
EPT1 = CPT1 * CHUNK
EP = EPT2 * 32
ER = EP // CHUNK
RPT = NP // NTILES
ZSUB = 32


def _dense_body(act, has_skip, x_ref, wt_ref, b_ref, *rest):
    if has_skip:
        z_ref, o_ref = rest
    else:
        (o_ref,) = rest
    o = jax.lax.dot_general(
        x_ref[...], wt_ref[...], (((1,), (0,)), ((), ())),
        preferred_element_type=jnp.float32,
        precision=jax.lax.Precision.HIGHEST)
    o = o + b_ref[...]
    if has_skip:
        o = o + z_ref[...]
    if act == "elu":
        o = jnp.where(o > 0, o, jnp.exp(jnp.minimum(o, 0.0)) - 1.0)
    o_ref[...] = o


def _dense(x, W, b, z=None, act="none", block=1024):
    n, din = x.shape
    dout = W.shape[0]
    wt = W.T
    b2 = b.reshape(1, dout)
    in_specs = [
        pl.BlockSpec((block, din), lambda i: (i, 0)),
        pl.BlockSpec((din, dout), lambda i: (0, 0)),
        pl.BlockSpec((1, dout), lambda i: (0, 0)),
    ]
    args = [x, wt, b2]
    if z is not None:
        in_specs.append(pl.BlockSpec((block, dout), lambda i: (i, 0)))
        args.append(z)
    return pl.pallas_call(
        functools.partial(_dense_body, act, z is not None),
        grid=(n // block,),
        in_specs=in_specs,
        out_specs=pl.BlockSpec((block, dout), lambda i: (i, 0)),
        out_shape=jax.ShapeDtypeStruct((n, dout), jnp.float32),
    )(*args)


def _combine_body(p_ref, xp_ref, o_ref):
    v = GAMMA * (p_ref[0] + p_ref[1]) + xp_ref[...]
    o_ref[...] = jnp.maximum(v, 0.0)


def _combine(p, xp, block=1024):
    spec = pl.BlockSpec((block, D), lambda i: (i, 0))
    return pl.pallas_call(
        _combine_body,
        grid=(NP // block,),
        in_specs=[pl.BlockSpec((2, block, D), lambda i: (0, i, 0)), spec],
        out_specs=spec,
        out_shape=jax.ShapeDtypeStruct((NP, D), jnp.float32),
    )(p, xp)


_MESH = plsc.VectorSubcoreMesh(core_axis_name="core", subcore_axis_name="subcore")

_SCRATCH = [
    pltpu.VMEM_SHARED((NP, D), jnp.float32),
    pltpu.VMEM((CHUNK, D), jnp.float32),
    pltpu.VMEM((CHUNK, D), jnp.float32),
    pltpu.VMEM((GB, CHUNK), jnp.int32),
    pltpu.VMEM((GB, CHUNK), jnp.int32),
    pltpu.VMEM((GB, CHUNK), jnp.int32),
    pltpu.VMEM((GB, CHUNK), jnp.float32),
    pltpu.VMEM((ZSUB, D), jnp.float32),
    pltpu.SemaphoreType.DMA,
    pltpu.SemaphoreType.DMA,
    pltpu.SemaphoreType.DMA,
    pltpu.SemaphoreType.DMA,
]

_GDN = lax.GatherDimensionNumbers(
    offset_dims=(), collapsed_slice_dims=(0,), start_index_map=(0,))


def _scale_rows(rows_v, wv_b, g):
    @pl.loop(0, CHUNK // LANES)
    def _(q):
        w16 = wv_b[g, pl.ds(q * LANES, LANES)]
        for ii in range(LANES):
            wb = lax.gather(w16, jnp.full((LANES, 1), ii, jnp.int32),
                            _GDN, (1,),
                            mode=lax.GatherScatterMode.PROMISE_IN_BOUNDS)
            i = q * LANES + ii
            for p in range(PIECES):
                sl = (i, pl.ds(p * LANES, LANES))
                rows_v[sl] = rows_v[sl] * wb


def _fill_zero(zero_v):
    @pl.loop(0, ZSUB)
    def _(i):
        for p in range(PIECES):
            zero_v[i, pl.ds(p * LANES, LANES)] = jnp.zeros((LANES,), jnp.float32)


def _zero_stripe(zero_v, acc_sh, t):
    for u in range(RPT // ZSUB):
        pltpu.sync_copy(zero_v, acc_sh.at[pl.ds(t * RPT + u * ZSUB, ZSUB)])


def _edge_batch(z_hbm, brow, gidx_b, didx_b, wv_b, acc_sh,
                rows0_v, rows1_v, sg0, sg1, ss0, ss1):
    @pl.loop(0, GB, step=2)
    def _(jj):
        g0 = jj
        g1 = jj + 1
        hg0 = pltpu.async_copy(z_hbm.at[gidx_b.at[g0]], rows0_v, sg0)
        hg1 = pltpu.async_copy(z_hbm.at[gidx_b.at[g1]], rows1_v, sg1)
        hg0.wait()
        _scale_rows(rows0_v, wv_b, g0)
        hs0 = pltpu.async_copy(rows0_v, acc_sh.at[didx_b.at[g0]], ss0, add=True)
        hg1.wait()
        _scale_rows(rows1_v, wv_b, g1)
        hs1 = pltpu.async_copy(rows1_v, acc_sh.at[didx_b.at[g1]], ss1, add=True)
        hs0.wait()
        hs1.wait()


def _ppr6_body(xps_hbm, src_hbm, dst_hbm, w_hbm, zbig_hbm,
               acc_sh, rows0_v, rows1_v, sidx_b, gidx_b, didx_b, wv_b, zero_v,
               sg0, sg1, ss0, ss1):
    c = lax.axis_index("core")
    t = lax.axis_index("subcore")
    slot0 = c * (K_ITERS + 1) * NP

    _fill_zero(zero_v)
    _zero_stripe(zero_v, acc_sh, t)
    for u in range(RPT // CHUNK):
        r0 = t * RPT + u * CHUNK
        pltpu.sync_copy(xps_hbm.at[pl.ds(c * NP + r0, CHUNK)], rows0_v)
        pltpu.sync_copy(rows0_v, zbig_hbm.at[pl.ds(slot0 + r0, CHUNK)])
    plsc.subcore_barrier()

    @pl.loop(0, K_ITERS)
    def _(k):
        gbase = slot0 + k * NP

        @pl.loop(0, NB1)
        def _(bb):
            brow = t * (CPT1 // GB) * GB + bb * GB
            pltpu.sync_copy(src_hbm.at[pl.ds(brow, GB)], sidx_b)
            pltpu.sync_copy(dst_hbm.at[pl.ds(brow, GB)], didx_b)
            pltpu.sync_copy(w_hbm.at[pl.ds(brow, GB)], wv_b)
            for r in range(GB):
                for q in range(CHUNK // LANES):
                    sl = (r, pl.ds(q * LANES, LANES))
                    gidx_b[sl] = sidx_b[sl] + gbase
            _edge_batch(zbig_hbm, brow, gidx_b, didx_b, wv_b, acc_sh,
                        rows0_v, rows1_v, sg0, sg1, ss0, ss1)

        plsc.subcore_barrier()

        wbase = gbase + NP
        for u in range(RPT // CHUNK):
            r0 = t * RPT + u * CHUNK
            h1 = pltpu.async_copy(acc_sh.at[pl.ds(r0, CHUNK)], rows1_v, sg1)
            h0 = pltpu.async_copy(
                zbig_hbm.at[pl.ds(slot0 + r0, CHUNK)], rows0_v, sg0)
            h1.wait()
            h0.wait()

            @pl.loop(0, CHUNK)
            def _(i):
                for p in range(PIECES):
                    sl = (i, pl.ds(p * LANES, LANES))
                    v = GAMMA * rows1_v[sl] + rows0_v[sl]
                    rows1_v[sl] = jnp.maximum(v, 0.0)

            pltpu.sync_copy(rows1_v, zbig_hbm.at[pl.ds(wbase + r0, CHUNK)])
            for v in range(CHUNK // ZSUB):
                pltpu.sync_copy(zero_v, acc_sh.at[pl.ds(r0 + v * ZSUB, ZSUB)])
        plsc.subcore_barrier()


def _ppr_layer1(xp, src2, dst2, w2):
    xps = jnp.concatenate([xp[:, :D], xp[:, D:]], axis=0)
    zshape = jax.ShapeDtypeStruct((2 * (K_ITERS + 1) * NP, D), jnp.float32)
    k = pl.kernel(_ppr6_body, out_type=zshape, mesh=_MESH,
                  scratch_types=_SCRATCH)
    zbig = k(xps, src2, dst2, w2)
    lo = K_ITERS * NP
    hi = (K_ITERS + 1) * NP
    return jnp.concatenate(
        [zbig[lo:hi], zbig[(K_ITERS + 1) * NP + lo:(K_ITERS + 1) * NP + hi]],
        axis=1)


def _spmm_body(z_hbm, src_hbm, dst_hbm, w_hbm, p_hbm,
               acc_sh, rows0_v, rows1_v, sidx_b, gidx_b, didx_b, wv_b, zero_v,
               sg0, sg1, ss0, ss1):
    c = lax.axis_index("core")
    t = lax.axis_index("subcore")

    _fill_zero(zero_v)
    _zero_stripe(zero_v, acc_sh, t)
    plsc.subcore_barrier()

    @pl.loop(0, NB2)
    def _(bb):
        brow = (c * NTILES + t) * (CPT2 // GB) * GB + bb * GB
        pltpu.sync_copy(src_hbm.at[pl.ds(brow, GB)], gidx_b)
        pltpu.sync_copy(dst_hbm.at[pl.ds(brow, GB)], didx_b)
        pltpu.sync_copy(w_hbm.at[pl.ds(brow, GB)], wv_b)
        _edge_batch(z_hbm, brow, gidx_b, didx_b, wv_b, acc_sh,
                    rows0_v, rows1_v, sg0, sg1, ss0, ss1)

    plsc.subcore_barrier()

    for u in range(RPT // CHUNK):
        r0 = t * RPT + u * CHUNK
        pltpu.sync_copy(acc_sh.at[pl.ds(r0, CHUNK)], rows0_v)
        pltpu.sync_copy(rows0_v, p_hbm.at[pl.ds(c * NP + r0, CHUNK)])


def _ppr_layer_iter(xp, src2, dst2, w2):
    pshape = jax.ShapeDtypeStruct((2 * NP, D), jnp.float32)
    spmm = pl.kernel(_spmm_body, out_type=pshape, mesh=_MESH,
                     scratch_types=_SCRATCH)
    z = xp
    for _ in range(K_ITERS):
        p = spmm(z, src2, dst2, w2)
        z = _combine(p.reshape(2, NP, D), xp)
    return z


def kernel(features, edge_index, edge_weight,
           W1, b1, W2, b2, W3, b3, W4, b4, W5, b5,
           VW0, Vb0, VW1, Vb1, VW2, Vb2, VW3, Vb3, VW, Vb):
    pad = EP - E
    dst2 = jnp.concatenate(
        [edge_index[0], jnp.zeros((pad,), jnp.int32)]).reshape(ER, CHUNK)
    src2 = jnp.concatenate(
        [edge_index[1], jnp.zeros((pad,), jnp.int32)]).reshape(ER, CHUNK)
    w2 = jnp.concatenate(
        [edge_weight, jnp.zeros((pad,), jnp.float32)]).reshape(ER, CHUNK)

    x = jnp.pad(features, ((0, NP - N), (0, 0)))
    z = _ppr_layer1(_dense(x, W1, b1), src2, dst2, w2)
    x = _dense(x, VW0, Vb0, z=z, act="elu")
    z = _ppr_layer_iter(_dense(x, W2, b2), src2, dst2, w2)
    x = _dense(x, VW1, Vb1, z=z, act="elu")
    z = _ppr_layer_iter(_dense(x, W3, b3), src2, dst2, w2)
    x = _dense(x, VW2, Vb2, z=z, act="elu")
    W4p = jnp.pad(W4, ((0, 64), (0, 0)))
    b4p = jnp.pad(b4, (0, 64))
    z = _ppr_layer_iter(_dense(x, W4p, b4p), src2, dst2, w2)[:, :64]
    x = _dense(x, VW3, Vb3, z=z, act="elu")
    W5p = jnp.pad(W5, ((0, 7), (0, 0)))
    b5p = jnp.pad(b5, (0, 7))
    VWp = jnp.pad(VW, ((0, 7), (0, 0)))
    Vbp = jnp.pad(Vb, (0, 7))
    z = _ppr_layer_iter(_dense(x, W5p, b5p), src2, dst2, w2)
    out = _dense(x, VWp, Vbp, z=z)[:N, :121]
    return (out, K_ITERS * 5)


# speedup vs baseline: 1.0010x; 1.0010x over previous
import functools

import jax
import jax.numpy as jnp
from jax import lax
from jax.experimental import pallas as pl
from jax.experimental.pallas import tpu as pltpu
from jax.experimental.pallas import tpu_sc as plsc

N = 10000
NP = 10240
E = 320000
GAMMA = 0.1
K_ITERS = 6
D = 128

NTILES = 16
LANES = 16
PIECES = D // LANES
CHUNK = 128
GB = 8
NB2 = 10
CPT2 = NB2 * GB
EPT2 = CPT2 * CHUNK
NB1 = 2 * NB2
CPT1 = NB1 * GB
EPT1 = CPT1 * CHUNK
EP = EPT2 * 32
ER = EP // CHUNK
RPT = NP // NTILES
ZSUB = 32


def _dense_body(act, has_skip, x_ref, wt_ref, b_ref, *rest):
    if has_skip:
        z_ref, o_ref = rest
    else:
        (o_ref,) = rest
    o = jax.lax.dot_general(
        x_ref[...], wt_ref[...], (((1,), (0,)), ((), ())),
        preferred_element_type=jnp.float32,
        precision=jax.lax.Precision.HIGHEST)
    o = o + b_ref[...]
    if has_skip:
        o = o + z_ref[...]
    if act == "elu":
        o = jnp.where(o > 0, o, jnp.exp(jnp.minimum(o, 0.0)) - 1.0)
    o_ref[...] = o


def _dense(x, W, b, z=None, act="none", block=1024):
    n, din = x.shape
    dout = W.shape[0]
    wt = W.T
    b2 = b.reshape(1, dout)
    in_specs = [
        pl.BlockSpec((block, din), lambda i: (i, 0)),
        pl.BlockSpec((din, dout), lambda i: (0, 0)),
        pl.BlockSpec((1, dout), lambda i: (0, 0)),
    ]
    args = [x, wt, b2]
    if z is not None:
        in_specs.append(pl.BlockSpec((block, dout), lambda i: (i, 0)))
        args.append(z)
    return pl.pallas_call(
        functools.partial(_dense_body, act, z is not None),
        grid=(n // block,),
        in_specs=in_specs,
        out_specs=pl.BlockSpec((block, dout), lambda i: (i, 0)),
        out_shape=jax.ShapeDtypeStruct((n, dout), jnp.float32),
    )(*args)


def _combine_body(p_ref, xp_ref, o_ref):
    v = GAMMA * (p_ref[0] + p_ref[1]) + xp_ref[...]
    o_ref[...] = jnp.maximum(v, 0.0)


def _combine(p, xp, block=1024):
    spec = pl.BlockSpec((block, D), lambda i: (i, 0))
    return pl.pallas_call(
        _combine_body,
        grid=(NP // block,),
        in_specs=[pl.BlockSpec((2, block, D), lambda i: (0, i, 0)), spec],
        out_specs=spec,
        out_shape=jax.ShapeDtypeStruct((NP, D), jnp.float32),
    )(p, xp)


_MESH = plsc.VectorSubcoreMesh(core_axis_name="core", subcore_axis_name="subcore")

_SCRATCH = [
    pltpu.VMEM_SHARED((NP, D), jnp.float32),
    pltpu.VMEM((CHUNK, D), jnp.float32),
    pltpu.VMEM((CHUNK, D), jnp.float32),
    pltpu.VMEM((GB, CHUNK), jnp.int32),
    pltpu.VMEM((GB, CHUNK), jnp.int32),
    pltpu.VMEM((GB, CHUNK), jnp.int32),
    pltpu.VMEM((GB, CHUNK), jnp.float32),
    pltpu.VMEM((ZSUB, D), jnp.float32),
    pltpu.SemaphoreType.DMA,
    pltpu.SemaphoreType.DMA,
    pltpu.SemaphoreType.DMA,
    pltpu.SemaphoreType.DMA,
]

_GDN = lax.GatherDimensionNumbers(
    offset_dims=(), collapsed_slice_dims=(0,), start_index_map=(0,))


def _scale_rows(rows_v, wv_b, g):
    @pl.loop(0, CHUNK // LANES)
    def _(q):
        w16 = wv_b[g, pl.ds(q * LANES, LANES)]
        for ii in range(LANES):
            wb = lax.gather(w16, jnp.full((LANES, 1), ii, jnp.int32),
                            _GDN, (1,),
                            mode=lax.GatherScatterMode.PROMISE_IN_BOUNDS)
            i = q * LANES + ii
            for p in range(PIECES):
                sl = (i, pl.ds(p * LANES, LANES))
                rows_v[sl] = rows_v[sl] * wb


def _fill_zero(zero_v):
    @pl.loop(0, ZSUB)
    def _(i):
        for p in range(PIECES):
            zero_v[i, pl.ds(p * LANES, LANES)] = jnp.zeros((LANES,), jnp.float32)


def _zero_stripe(zero_v, acc_sh, t):
    for u in range(RPT // ZSUB):
        pltpu.sync_copy(zero_v, acc_sh.at[pl.ds(t * RPT + u * ZSUB, ZSUB)])


def _edge_batch(z_hbm, brow, gidx_b, didx_b, wv_b, acc_sh,
                rows0_v, rows1_v, sg0, sg1, ss0, ss1):
    half = CHUNK // 2

    def fire(g, buf, sem):
        ha = pltpu.async_copy(
            z_hbm.at[gidx_b.at[g, pl.ds(0, half)]],
            buf.at[pl.ds(0, half)], sem)
        hb = pltpu.async_copy(
            z_hbm.at[gidx_b.at[g, pl.ds(half, half)]],
            buf.at[pl.ds(half, half)], sem)
        return ha, hb

    @pl.loop(0, GB, step=2)
    def _(jj):
        g0 = jj
        g1 = jj + 1
        h0a, h0b = fire(g0, rows0_v, sg0)
        h1a, h1b = fire(g1, rows1_v, sg1)
        h0a.wait()
        h0b.wait()
        _scale_rows(rows0_v, wv_b, g0)
        hs0 = pltpu.async_copy(rows0_v, acc_sh.at[didx_b.at[g0]], ss0, add=True)
        h1a.wait()
        h1b.wait()
        _scale_rows(rows1_v, wv_b, g1)
        hs1 = pltpu.async_copy(rows1_v, acc_sh.at[didx_b.at[g1]], ss1, add=True)
        hs0.wait()
        hs1.wait()


def _ppr6_body(xps_hbm, src_hbm, dst_hbm, w_hbm, zbig_hbm,
               acc_sh, rows0_v, rows1_v, sidx_b, gidx_b, didx_b, wv_b, zero_v,
               sg0, sg1, ss0, ss1):
    c = lax.axis_index("core")
    t = lax.axis_index("subcore")
    slot0 = c * (K_ITERS + 1) * NP

    _fill_zero(zero_v)
    _zero_stripe(zero_v, acc_sh, t)
    for u in range(RPT // CHUNK):
        r0 = t * RPT + u * CHUNK
        pltpu.sync_copy(xps_hbm.at[pl.ds(c * NP + r0, CHUNK)], rows0_v)
        pltpu.sync_copy(rows0_v, zbig_hbm.at[pl.ds(slot0 + r0, CHUNK)])
    plsc.subcore_barrier()

    @pl.loop(0, K_ITERS)
    def _(k):
        gbase = slot0 + k * NP

        @pl.loop(0, NB1)
        def _(bb):
            brow = t * (CPT1 // GB) * GB + bb * GB
            pltpu.sync_copy(src_hbm.at[pl.ds(brow, GB)], sidx_b)
            pltpu.sync_copy(dst_hbm.at[pl.ds(brow, GB)], didx_b)
            pltpu.sync_copy(w_hbm.at[pl.ds(brow, GB)], wv_b)
            for r in range(GB):
                for q in range(CHUNK // LANES):
                    sl = (r, pl.ds(q * LANES, LANES))
                    gidx_b[sl] = sidx_b[sl] + gbase
            _edge_batch(zbig_hbm, brow, gidx_b, didx_b, wv_b, acc_sh,
                        rows0_v, rows1_v, sg0, sg1, ss0, ss1)

        plsc.subcore_barrier()

        wbase = gbase + NP
        for u in range(RPT // CHUNK):
            r0 = t * RPT + u * CHUNK
            h1 = pltpu.async_copy(acc_sh.at[pl.ds(r0, CHUNK)], rows1_v, sg1)
            h0 = pltpu.async_copy(
                zbig_hbm.at[pl.ds(slot0 + r0, CHUNK)], rows0_v, sg0)
            h1.wait()
            h0.wait()

            @pl.loop(0, CHUNK)
            def _(i):
                for p in range(PIECES):
                    sl = (i, pl.ds(p * LANES, LANES))
                    v = GAMMA * rows1_v[sl] + rows0_v[sl]
                    rows1_v[sl] = jnp.maximum(v, 0.0)

            pltpu.sync_copy(rows1_v, zbig_hbm.at[pl.ds(wbase + r0, CHUNK)])
            for v in range(CHUNK // ZSUB):
                pltpu.sync_copy(zero_v, acc_sh.at[pl.ds(r0 + v * ZSUB, ZSUB)])
        plsc.subcore_barrier()


def _ppr_layer1(xp, src2, dst2, w2):
    xps = jnp.concatenate([xp[:, :D], xp[:, D:]], axis=0)
    zshape = jax.ShapeDtypeStruct((2 * (K_ITERS + 1) * NP, D), jnp.float32)
    k = pl.kernel(_ppr6_body, out_type=zshape, mesh=_MESH,
                  scratch_types=_SCRATCH)
    zbig = k(xps, src2, dst2, w2)
    lo = K_ITERS * NP
    hi = (K_ITERS + 1) * NP
    return jnp.concatenate(
        [zbig[lo:hi], zbig[(K_ITERS + 1) * NP + lo:(K_ITERS + 1) * NP + hi]],
        axis=1)


def _spmm_body(z_hbm, src_hbm, dst_hbm, w_hbm, p_hbm,
               acc_sh, rows0_v, rows1_v, sidx_b, gidx_b, didx_b, wv_b, zero_v,
               sg0, sg1, ss0, ss1):
    c = lax.axis_index("core")
    t = lax.axis_index("subcore")

    _fill_zero(zero_v)
    _zero_stripe(zero_v, acc_sh, t)
    plsc.subcore_barrier()

    @pl.loop(0, NB2)
    def _(bb):
        brow = (c * NTILES + t) * (CPT2 // GB) * GB + bb * GB
        pltpu.sync_copy(src_hbm.at[pl.ds(brow, GB)], gidx_b)
        pltpu.sync_copy(dst_hbm.at[pl.ds(brow, GB)], didx_b)
        pltpu.sync_copy(w_hbm.at[pl.ds(brow, GB)], wv_b)
        _edge_batch(z_hbm, brow, gidx_b, didx_b, wv_b, acc_sh,
                    rows0_v, rows1_v, sg0, sg1, ss0, ss1)

    plsc.subcore_barrier()

    for u in range(RPT // CHUNK):
        r0 = t * RPT + u * CHUNK
        pltpu.sync_copy(acc_sh.at[pl.ds(r0, CHUNK)], rows0_v)
        pltpu.sync_copy(rows0_v, p_hbm.at[pl.ds(c * NP + r0, CHUNK)])


def _ppr_layer_iter(xp, src2, dst2, w2):
    pshape = jax.ShapeDtypeStruct((2 * NP, D), jnp.float32)
    spmm = pl.kernel(_spmm_body, out_type=pshape, mesh=_MESH,
                     scratch_types=_SCRATCH)
    z = xp
    for _ in range(K_ITERS):
        p = spmm(z, src2, dst2, w2)
        z = _combine(p.reshape(2, NP, D), xp)
    return z


def kernel(features, edge_index, edge_weight,
           W1, b1, W2, b2, W3, b3, W4, b4, W5, b5,
           VW0, Vb0, VW1, Vb1, VW2, Vb2, VW3, Vb3, VW, Vb):
    pad = EP - E
    dst2 = jnp.concatenate(
        [edge_index[0], jnp.zeros((pad,), jnp.int32)]).reshape(ER, CHUNK)
    src2 = jnp.concatenate(
        [edge_index[1], jnp.zeros((pad,), jnp.int32)]).reshape(ER, CHUNK)
    w2 = jnp.concatenate(
        [edge_weight, jnp.zeros((pad,), jnp.float32)]).reshape(ER, CHUNK)

    x = jnp.pad(features, ((0, NP - N), (0, 0)))
    z = _ppr_layer1(_dense(x, W1, b1), src2, dst2, w2)
    x = _dense(x, VW0, Vb0, z=z, act="elu")
    z = _ppr_layer_iter(_dense(x, W2, b2), src2, dst2, w2)
    x = _dense(x, VW1, Vb1, z=z, act="elu")
    z = _ppr_layer_iter(_dense(x, W3, b3), src2, dst2, w2)
    x = _dense(x, VW2, Vb2, z=z, act="elu")
    W4p = jnp.pad(W4, ((0, 64), (0, 0)))
    b4p = jnp.pad(b4, (0, 64))
    z = _ppr_layer_iter(_dense(x, W4p, b4p), src2, dst2, w2)[:, :64]
    x = _dense(x, VW3, Vb3, z=z, act="elu")
    W5p = jnp.pad(W5, ((0, 7), (0, 0)))
    b5p = jnp.pad(b5, (0, 7))
    VWp = jnp.pad(VW, ((0, 7), (0, 0)))
    Vbp = jnp.pad(Vb, (0, 7))
    z = _ppr_layer_iter(_dense(x, W5p, b5p), src2, dst2, w2)
    out = _dense(x, VWp, Vbp, z=z)[:N, :121]
    return (out, K_ITERS * 5)
